# baseline (speedup 1.0000x reference)
import functools

import jax
import jax.numpy as jnp
import numpy as np
from jax import lax
from jax.experimental import pallas as pl
from jax.experimental.pallas import tpu as pltpu
from jax.experimental.pallas import tpu_sc as plsc

N = 10000
E = 320000
F = 128
H = 8
D = 16
ACC_W = 144

NC = 2
NS = 16
NW = NC * NS
EW = E // NW
C = 40
NCHUNK = EW // C
IB = 25
NBLK = NCHUNK // IB
N_PAD = 10000
ROWS_PER_SUB = N_PAD // NS


def _proj_body(x_ref, w_ref, b_ref, q_ref, k_ref, v_ref):
    y = jnp.dot(x_ref[...], w_ref[...], preferred_element_type=jnp.float32)
    y = y + b_ref[...]
    q_ref[...] = y[:, :F].astype(jnp.bfloat16)
    k_ref[...] = y[:, F:2 * F].astype(jnp.bfloat16)
    v_ref[...] = y[:, 2 * F:].astype(jnp.bfloat16)


def _project(xin, w_all, b_all, blk=1000):
    return pl.pallas_call(
        _proj_body,
        grid=(N // blk,),
        in_specs=[
            pl.BlockSpec((blk, F), lambda i: (i, 0)),
            pl.BlockSpec((F, 3 * F), lambda i: (0, 0)),
            pl.BlockSpec((1, 3 * F), lambda i: (0, 0)),
        ],
        out_specs=[pl.BlockSpec((blk, F), lambda i: (i, 0))] * 3,
        out_shape=[jax.ShapeDtypeStruct((N, F), jnp.bfloat16)] * 3,
    )(xin, w_all, b_all)


_DNUMS = lax.GatherDimensionNumbers(
    offset_dims=(), collapsed_slice_dims=(0,), start_index_map=(0,))


def _take(x, idx):
    return lax.gather(x, idx[:, None], _DNUMS, slice_sizes=(1,),
                      mode=lax.GatherScatterMode.PROMISE_IN_BOUNDS)


def _edge_body(q_hbm, k_hbm, v_hbm, src_hbm, dst_hbm, out_hbm,
               sblk, dblk, qb0, kb0, vb0, qb1, kb1, vb1,
               msg, prod, acc, sem0, sem1):
    c = lax.axis_index("c")
    s = lax.axis_index("s")
    wid = s * NC + c
    qb = (qb0, qb1)
    kb = (kb0, kb1)
    vb = (vb0, vb1)
    sem = (sem0, sem1)
    zvec = jnp.zeros((16,), jnp.float32)
    iota = lax.iota(jnp.int32, 16)
    perm8 = iota ^ 8
    base_l = (iota >> 3) * 128 + ((iota & 7) >> 1) * 32 + (iota & 1) * 8

    def zrow(r, carry):
        for c9 in range(ACC_W // 16):
            msg[r, pl.ds(c9 * 16, 16)] = zvec
        return carry
    lax.fori_loop(0, C, zrow, None)
    for j in range(ROWS_PER_SUB // C):
        pltpu.sync_copy(msg, acc.at[pl.ds(s * ROWS_PER_SUB + j * C, C)])
    pltpu.sync_copy(msg.at[pl.ds(0, ROWS_PER_SUB % C)],
                    acc.at[pl.ds(s * ROWS_PER_SUB + (ROWS_PER_SUB // C) * C,
                                 ROWS_PER_SUB % C)])
    plsc.subcore_barrier()

    chunk_base = wid * NCHUNK

    def issue(jj, b):
        pltpu.async_copy(q_hbm.at[dblk.at[jj]], qb[b], sem[b])
        pltpu.async_copy(k_hbm.at[sblk.at[jj]], kb[b], sem[b])
        pltpu.async_copy(v_hbm.at[sblk.at[jj]], vb[b], sem[b])

    def process(jj, b):
        pltpu.make_async_copy(q_hbm.at[pl.ds(0, C)], qb[b], sem[b]).wait()
        pltpu.make_async_copy(k_hbm.at[pl.ds(0, C)], kb[b], sem[b]).wait()
        pltpu.make_async_copy(v_hbm.at[pl.ds(0, C)], vb[b], sem[b]).wait()

        @plsc.parallel_loop(0, C // 2, 1, unroll=4)
        def pair(p2):
            e0 = 2 * p2
            e1 = e0 + 1
            o = (p2 & 15) * 256
            def halves(ref, e, h2):
                w = plsc.bitcast(ref[e, pl.ds(h2 * 32, 32)], jnp.int32)
                lo = plsc.bitcast(w << 16, jnp.float32)
                hi = plsc.bitcast(w & jnp.int32(-65536), jnp.float32)
                return lo, hi
            for e, ebase in ((e0, 0), (e1, 128)):
                for h2 in range(4):
                    qlo, qhi = halves(qb[b], e, h2)
                    klo, khi = halves(kb[b], e, h2)
                    prod[pl.ds(o + ebase + 32 * h2, 16)] = qlo * klo
                    prod[pl.ds(o + ebase + 32 * h2 + 16, 16)] = qhi * khi
            gs = [plsc.load_gather(prod, [base_l + (o + (d & 1) * 16 + (d >> 1))])
                  for d in range(16)]
            while len(gs) > 1:
                gs = [gs[i] + gs[i + 1] for i in range(0, len(gs), 2)]
            alpha = jnp.exp(gs[0])
            for e, abase in ((e0, 0), (e1, 8)):
                for h2 in range(4):
                    vlo, vhi = halves(vb[b], e, h2)
                    alo = _take(alpha, jnp.full((16,), abase + 2 * h2,
                                                jnp.int32))
                    ahi = _take(alpha, jnp.full((16,), abase + 2 * h2 + 1,
                                                jnp.int32))
                    msg[e, pl.ds(32 * h2, 16)] = vlo * alo
                    msg[e, pl.ds(32 * h2 + 16, 16)] = vhi * ahi
            msg[e0, pl.ds(128, 16)] = alpha
            msg[e1, pl.ds(128, 16)] = _take(alpha, perm8)

        pltpu.sync_copy(msg, acc.at[dblk.at[jj]], add=True)

    def block(bi, carry):
        row0 = chunk_base + bi * IB
        pltpu.sync_copy(src_hbm.at[pl.ds(row0, IB)], sblk)
        pltpu.sync_copy(dst_hbm.at[pl.ds(row0, IB)], dblk)
        issue(0, 0)

        def chunk2(i, inner):
            j0 = 2 * i
            issue(j0 + 1, 1)
            process(j0, 0)
            issue(j0 + 2, 0)
            process(j0 + 1, 1)
            return inner

        lax.fori_loop(0, (IB - 1) // 2, chunk2, None)
        process(IB - 1, 0)
        return carry

    lax.fori_loop(0, NBLK, block, None)
    plsc.subcore_barrier()

    pltpu.sync_copy(acc.at[pl.ds(s * ROWS_PER_SUB, ROWS_PER_SUB)],
                    out_hbm.at[c, pl.ds(s * ROWS_PER_SUB, ROWS_PER_SUB)])


def _edge(q_t, k_t, v_t, src, dst):
    fn = functools.partial(
        pl.kernel,
        out_type=jax.ShapeDtypeStruct((NC, N_PAD, ACC_W), jnp.float32),
        mesh=plsc.VectorSubcoreMesh(core_axis_name="c", subcore_axis_name="s"),
        compiler_params=pltpu.CompilerParams(
            use_tc_tiling_on_sc=False, needs_layout_passes=False),
        scratch_types=(
            [pltpu.VMEM((IB, C), jnp.int32),
             pltpu.VMEM((IB, C), jnp.int32)]
            + [pltpu.VMEM((C, F), jnp.bfloat16)] * 6
            + [
                pltpu.VMEM((C, ACC_W), jnp.float32),
                pltpu.VMEM((16 * 256,), jnp.float32),
                pltpu.VMEM_SHARED((N_PAD, ACC_W), jnp.float32),
                pltpu.SemaphoreType.DMA,
                pltpu.SemaphoreType.DMA,
            ]),
    )(_edge_body)
    return fn(q_t, k_t, v_t, src, dst)


def _post_body(relu, part_ref, x_ref, rp_ref, wa_ref, ba_ref, skip_ref, o_ref):
    a = part_ref[0] + part_ref[1]
    den = jnp.dot(a, rp_ref[...], preferred_element_type=jnp.float32) + 1e-16
    att = a[:, :F] / den
    g = jax.nn.gelu(att)
    o = jnp.dot(g, wa_ref[...], preferred_element_type=jnp.float32) + ba_ref[...]
    beta = jax.nn.sigmoid(skip_ref[0, 0])
    o = beta * o + (1.0 - beta) * x_ref[...]
    if relu:
        o = jnp.maximum(o, 0.0)
    o_ref[...] = o


_QPERM = np.zeros(F, np.int32)
for _h2 in range(4):
    for _w in range(32):
        _QPERM[32 * _h2 + _w] = 32 * _h2 + (
            _w // 2 if _w % 2 == 0 else 16 + (_w - 1) // 2)


def _post(partials, xin, wa, ba, skip, relu, blk=1000):
    rp = np.zeros((ACC_W, F), np.float32)
    for h in range(H):
        rp[F + h, h * D:(h + 1) * D] = 1.0
    return pl.pallas_call(
        functools.partial(_post_body, relu),
        grid=(N // blk,),
        in_specs=[
            pl.BlockSpec((NC, blk, ACC_W), lambda i: (0, i, 0)),
            pl.BlockSpec((blk, F), lambda i: (i, 0)),
            pl.BlockSpec((ACC_W, F), lambda i: (0, 0)),
            pl.BlockSpec((F, F), lambda i: (0, 0)),
            pl.BlockSpec((1, F), lambda i: (0, 0)),
            pl.BlockSpec((1, 1), lambda i: (0, 0)),
        ],
        out_specs=pl.BlockSpec((blk, F), lambda i: (i, 0)),
        out_shape=jax.ShapeDtypeStruct((N, F), jnp.float32),
    )(partials, xin, jnp.asarray(rp), wa, ba.reshape(1, F),
      skip.reshape(1, 1))


def _fold(p):
    scale = p["p_rel"] / np.sqrt(float(D))
    wq = p["Wq"].reshape(F, H, D) * scale[None, :, None]
    bq = p["bq"].reshape(H, D) * scale[:, None]
    wk = jnp.einsum("fhd,hde->fhe", p["Wk"].reshape(F, H, D), p["a_rel"])
    bk = jnp.einsum("hd,hde->he", p["bk"].reshape(H, D), p["a_rel"])
    wv = jnp.einsum("fhd,hde->fhe", p["Wv"].reshape(F, H, D), p["m_rel"])
    bv = jnp.einsum("hd,hde->he", p["bv"].reshape(H, D), p["m_rel"])
    w_all = jnp.concatenate(
        [wq.reshape(F, F), wk.reshape(F, F), wv.reshape(F, F)[:, _QPERM]],
        axis=1)
    b_all = jnp.concatenate(
        [bq.reshape(F), bk.reshape(F),
         bv.reshape(F)[_QPERM]]).reshape(1, 3 * F)
    return w_all, b_all


def kernel(x, edge_index, params):
    src = edge_index[0].astype(jnp.int32).reshape(E // C, C)
    dst = edge_index[1].astype(jnp.int32).reshape(E // C, C)
    h = x
    for li, name in enumerate(("l1", "l2")):
        p = params[name]
        w_all, b_all = _fold(p)
        q_t, k_t, v_t = _project(h, w_all, b_all)
        partials = _edge(q_t, k_t, v_t, src, dst)
        h = _post(partials, h, p["Wa"], p["ba"], p["skip"], relu=(li == 0))
    return h


# speedup vs baseline: 1.0934x; 1.0934x over previous
import functools

import jax
import jax.numpy as jnp
import numpy as np
from jax import lax
from jax.experimental import pallas as pl
from jax.experimental.pallas import tpu as pltpu
from jax.experimental.pallas import tpu_sc as plsc

N = 10000
E = 320000
F = 128
H = 8
D = 16
ACC_W = 144

NC = 2
NS = 16
NW = NC * NS
EW = E // NW
C = 40
NCHUNK = EW // C
IB = 25
NBLK = NCHUNK // IB
N_PAD = 10000
ROWS_PER_SUB = N_PAD // NS


def _proj_body(x_ref, w_ref, b_ref, q_ref, k_ref, v_ref):
    y = jnp.dot(x_ref[...], w_ref[...], preferred_element_type=jnp.float32)
    y = y + b_ref[...]
    q_ref[...] = y[:, :F].astype(jnp.bfloat16)
    k_ref[...] = y[:, F:2 * F].astype(jnp.bfloat16)
    v_ref[...] = y[:, 2 * F:].astype(jnp.bfloat16)


def _project(xin, w_all, b_all, blk=1000):
    return pl.pallas_call(
        _proj_body,
        grid=(N // blk,),
        in_specs=[
            pl.BlockSpec((blk, F), lambda i: (i, 0)),
            pl.BlockSpec((F, 3 * F), lambda i: (0, 0)),
            pl.BlockSpec((1, 3 * F), lambda i: (0, 0)),
        ],
        out_specs=[pl.BlockSpec((blk, F), lambda i: (i, 0))] * 3,
        out_shape=[jax.ShapeDtypeStruct((N, F), jnp.bfloat16)] * 3,
    )(xin, w_all, b_all)


_DNUMS = lax.GatherDimensionNumbers(
    offset_dims=(), collapsed_slice_dims=(0,), start_index_map=(0,))


def _take(x, idx):
    return lax.gather(x, idx[:, None], _DNUMS, slice_sizes=(1,),
                      mode=lax.GatherScatterMode.PROMISE_IN_BOUNDS)


def _edge_body(q_hbm, k_hbm, v_hbm, src_hbm, dst_hbm, out_hbm,
               sblk, dblk, qb0, kb0, vb0, qb1, kb1, vb1,
               msg, prod, acc, sem0, sem1):
    c = lax.axis_index("c")
    s = lax.axis_index("s")
    wid = s * NC + c
    qb = (qb0, qb1)
    kb = (kb0, kb1)
    vb = (vb0, vb1)
    sem = (sem0, sem1)
    zvec = jnp.zeros((16,), jnp.float32)
    iota = lax.iota(jnp.int32, 16)
    perm8 = iota ^ 8
    base_l = (iota >> 3) * 128 + ((iota & 7) >> 1) * 32 + (iota & 1) * 8

    def zrow(r, carry):
        for c9 in range(ACC_W // 16):
            msg[r, pl.ds(c9 * 16, 16)] = zvec
        return carry
    lax.fori_loop(0, C, zrow, None)
    for j in range(ROWS_PER_SUB // C):
        pltpu.sync_copy(msg, acc.at[pl.ds(s * ROWS_PER_SUB + j * C, C)])
    pltpu.sync_copy(msg.at[pl.ds(0, ROWS_PER_SUB % C)],
                    acc.at[pl.ds(s * ROWS_PER_SUB + (ROWS_PER_SUB // C) * C,
                                 ROWS_PER_SUB % C)])
    plsc.subcore_barrier()

    chunk_base = wid * NCHUNK

    def issue(jj, b):
        pltpu.async_copy(q_hbm.at[dblk.at[jj]], qb[b], sem[b])
        pltpu.async_copy(k_hbm.at[sblk.at[jj]], kb[b], sem[b])
        pltpu.async_copy(v_hbm.at[sblk.at[jj]], vb[b], sem[b])

    def process(jj, b):
        pltpu.make_async_copy(q_hbm.at[pl.ds(0, C)], qb[b], sem[b]).wait()
        pltpu.make_async_copy(k_hbm.at[pl.ds(0, C)], kb[b], sem[b]).wait()
        pltpu.make_async_copy(v_hbm.at[pl.ds(0, C)], vb[b], sem[b]).wait()

        @plsc.parallel_loop(0, C // 2, 1, unroll=2)
        def pair(p2):
            e0 = 2 * p2
            e1 = e0 + 1
            o = (p2 & 15) * 256
            def halves(ref, e, h2):
                w = plsc.bitcast(ref[e, pl.ds(h2 * 32, 32)], jnp.int32)
                lo = plsc.bitcast(w << 16, jnp.float32)
                hi = plsc.bitcast(w & jnp.int32(-65536), jnp.float32)
                return lo, hi
            for e, ebase in ((e0, 0), (e1, 128)):
                for h2 in range(4):
                    qlo, qhi = halves(qb[b], e, h2)
                    klo, khi = halves(kb[b], e, h2)
                    prod[pl.ds(o + ebase + 32 * h2, 16)] = qlo * klo
                    prod[pl.ds(o + ebase + 32 * h2 + 16, 16)] = qhi * khi
            gs = [plsc.load_gather(prod, [base_l + (o + (d & 1) * 16 + (d >> 1))])
                  for d in range(16)]
            while len(gs) > 1:
                gs = [gs[i] + gs[i + 1] for i in range(0, len(gs), 2)]
            alpha = jnp.exp(gs[0])
            for e, abase in ((e0, 0), (e1, 8)):
                for h2 in range(4):
                    vlo, vhi = halves(vb[b], e, h2)
                    alo = _take(alpha, jnp.full((16,), abase + 2 * h2,
                                                jnp.int32))
                    ahi = _take(alpha, jnp.full((16,), abase + 2 * h2 + 1,
                                                jnp.int32))
                    msg[e, pl.ds(32 * h2, 16)] = vlo * alo
                    msg[e, pl.ds(32 * h2 + 16, 16)] = vhi * ahi
            msg[e0, pl.ds(128, 16)] = alpha
            msg[e1, pl.ds(128, 16)] = _take(alpha, perm8)

        pltpu.sync_copy(msg, acc.at[dblk.at[jj]], add=True)

    def block(bi, carry):
        row0 = chunk_base + bi * IB
        pltpu.sync_copy(src_hbm.at[pl.ds(row0, IB)], sblk)
        pltpu.sync_copy(dst_hbm.at[pl.ds(row0, IB)], dblk)
        issue(0, 0)

        def chunk2(i, inner):
            j0 = 2 * i
            issue(j0 + 1, 1)
            process(j0, 0)
            issue(j0 + 2, 0)
            process(j0 + 1, 1)
            return inner

        lax.fori_loop(0, (IB - 1) // 2, chunk2, None)
        process(IB - 1, 0)
        return carry

    lax.fori_loop(0, NBLK, block, None)
    plsc.subcore_barrier()

    pltpu.sync_copy(acc.at[pl.ds(s * ROWS_PER_SUB, ROWS_PER_SUB)],
                    out_hbm.at[c, pl.ds(s * ROWS_PER_SUB, ROWS_PER_SUB)])


def _edge(q_t, k_t, v_t, src, dst):
    fn = functools.partial(
        pl.kernel,
        out_type=jax.ShapeDtypeStruct((NC, N_PAD, ACC_W), jnp.float32),
        mesh=plsc.VectorSubcoreMesh(core_axis_name="c", subcore_axis_name="s"),
        compiler_params=pltpu.CompilerParams(
            use_tc_tiling_on_sc=False, needs_layout_passes=False),
        scratch_types=(
            [pltpu.VMEM((IB, C), jnp.int32),
             pltpu.VMEM((IB, C), jnp.int32)]
            + [pltpu.VMEM((C, F), jnp.bfloat16)] * 6
            + [
                pltpu.VMEM((C, ACC_W), jnp.float32),
                pltpu.VMEM((16 * 256,), jnp.float32),
                pltpu.VMEM_SHARED((N_PAD, ACC_W), jnp.float32),
                pltpu.SemaphoreType.DMA,
                pltpu.SemaphoreType.DMA,
            ]),
    )(_edge_body)
    return fn(q_t, k_t, v_t, src, dst)


def _post_body(relu, part_ref, x_ref, rp_ref, wa_ref, ba_ref, skip_ref, o_ref):
    a = part_ref[0] + part_ref[1]
    den = jnp.dot(a, rp_ref[...], preferred_element_type=jnp.float32) + 1e-16
    att = a[:, :F] / den
    g = jax.nn.gelu(att)
    o = jnp.dot(g, wa_ref[...], preferred_element_type=jnp.float32) + ba_ref[...]
    beta = jax.nn.sigmoid(skip_ref[0, 0])
    o = beta * o + (1.0 - beta) * x_ref[...]
    if relu:
        o = jnp.maximum(o, 0.0)
    o_ref[...] = o


_QPERM = np.zeros(F, np.int32)
for _h2 in range(4):
    for _w in range(32):
        _QPERM[32 * _h2 + _w] = 32 * _h2 + (
            _w // 2 if _w % 2 == 0 else 16 + (_w - 1) // 2)


def _post(partials, xin, wa, ba, skip, relu, blk=1000):
    rp = np.zeros((ACC_W, F), np.float32)
    for h in range(H):
        rp[F + h, h * D:(h + 1) * D] = 1.0
    return pl.pallas_call(
        functools.partial(_post_body, relu),
        grid=(N // blk,),
        in_specs=[
            pl.BlockSpec((NC, blk, ACC_W), lambda i: (0, i, 0)),
            pl.BlockSpec((blk, F), lambda i: (i, 0)),
            pl.BlockSpec((ACC_W, F), lambda i: (0, 0)),
            pl.BlockSpec((F, F), lambda i: (0, 0)),
            pl.BlockSpec((1, F), lambda i: (0, 0)),
            pl.BlockSpec((1, 1), lambda i: (0, 0)),
        ],
        out_specs=pl.BlockSpec((blk, F), lambda i: (i, 0)),
        out_shape=jax.ShapeDtypeStruct((N, F), jnp.float32),
    )(partials, xin, jnp.asarray(rp), wa, ba.reshape(1, F),
      skip.reshape(1, 1))


def _fold(p):
    scale = p["p_rel"] / np.sqrt(float(D))
    wq = p["Wq"].reshape(F, H, D) * scale[None, :, None]
    bq = p["bq"].reshape(H, D) * scale[:, None]
    wk = jnp.einsum("fhd,hde->fhe", p["Wk"].reshape(F, H, D), p["a_rel"])
    bk = jnp.einsum("hd,hde->he", p["bk"].reshape(H, D), p["a_rel"])
    wv = jnp.einsum("fhd,hde->fhe", p["Wv"].reshape(F, H, D), p["m_rel"])
    bv = jnp.einsum("hd,hde->he", p["bv"].reshape(H, D), p["m_rel"])
    w_all = jnp.concatenate(
        [wq.reshape(F, F), wk.reshape(F, F), wv.reshape(F, F)[:, _QPERM]],
        axis=1)
    b_all = jnp.concatenate(
        [bq.reshape(F), bk.reshape(F),
         bv.reshape(F)[_QPERM]]).reshape(1, 3 * F)
    return w_all, b_all


def kernel(x, edge_index, params):
    src = edge_index[0].astype(jnp.int32).reshape(E // C, C)
    dst = edge_index[1].astype(jnp.int32).reshape(E // C, C)
    h = x
    for li, name in enumerate(("l1", "l2")):
        p = params[name]
        w_all, b_all = _fold(p)
        q_t, k_t, v_t = _project(h, w_all, b_all)
        partials = _edge(q_t, k_t, v_t, src, dst)
        h = _post(partials, h, p["Wa"], p["ba"], p["skip"], relu=(li == 0))
    return h


# speedup vs baseline: 1.4079x; 1.2876x over previous
import functools

import jax
import jax.numpy as jnp
import numpy as np
from jax import lax
from jax.experimental import pallas as pl
from jax.experimental.pallas import tpu as pltpu
from jax.experimental.pallas import tpu_sc as plsc

N = 10000
E = 320000
F = 128
H = 8
D = 16
ACC_W = 144

NC = 2
NS = 16
NW = NC * NS
EW = E // NW
C = 40
NCHUNK = EW // C
IB = 25
NBLK = NCHUNK // IB
N_PAD = 10000
ROWS_PER_SUB = N_PAD // NS


def _proj_body(x_ref, w_ref, b_ref, q_ref, k_ref, v_ref):
    y = jnp.dot(x_ref[...], w_ref[...], preferred_element_type=jnp.float32)
    y = y + b_ref[...]
    q_ref[...] = y[:, :F]
    k_ref[...] = y[:, F:2 * F]
    v_ref[...] = y[:, 2 * F:]


def _project(xin, w_all, b_all, blk=1000):
    return pl.pallas_call(
        _proj_body,
        grid=(N // blk,),
        in_specs=[
            pl.BlockSpec((blk, F), lambda i: (i, 0)),
            pl.BlockSpec((F, 3 * F), lambda i: (0, 0)),
            pl.BlockSpec((1, 3 * F), lambda i: (0, 0)),
        ],
        out_specs=[pl.BlockSpec((blk, F), lambda i: (i, 0))] * 3,
        out_shape=[jax.ShapeDtypeStruct((N, F), jnp.float32)] * 3,
    )(xin, w_all, b_all)


_DNUMS = lax.GatherDimensionNumbers(
    offset_dims=(), collapsed_slice_dims=(0,), start_index_map=(0,))


def _take(x, idx):
    return lax.gather(x, idx[:, None], _DNUMS, slice_sizes=(1,),
                      mode=lax.GatherScatterMode.PROMISE_IN_BOUNDS)


def _edge_body(q_hbm, k_hbm, v_hbm, src_hbm, dst_hbm, out_hbm,
               sblk, dblk, qb0, kb0, vb0, qb1, kb1, vb1,
               msg, prod, acc, sem0, sem1):
    c = lax.axis_index("c")
    s = lax.axis_index("s")
    wid = s * NC + c
    qb = (qb0, qb1)
    kb = (kb0, kb1)
    vb = (vb0, vb1)
    sem = (sem0, sem1)
    zvec = jnp.zeros((16,), jnp.float32)
    iota = lax.iota(jnp.int32, 16)
    perm8 = iota ^ 8
    base_l = (iota >> 3) * 128 + ((iota & 7) >> 1) * 32 + (iota & 1) * 8

    def zrow(r, carry):
        for c9 in range(ACC_W // 16):
            msg[r, pl.ds(c9 * 16, 16)] = zvec
        return carry
    lax.fori_loop(0, C, zrow, None)
    for j in range(ROWS_PER_SUB // C):
        pltpu.sync_copy(msg, acc.at[pl.ds(s * ROWS_PER_SUB + j * C, C)])
    pltpu.sync_copy(msg.at[pl.ds(0, ROWS_PER_SUB % C)],
                    acc.at[pl.ds(s * ROWS_PER_SUB + (ROWS_PER_SUB // C) * C,
                                 ROWS_PER_SUB % C)])
    plsc.subcore_barrier()

    chunk_base = wid * NCHUNK

    def issue(jj, b):
        pltpu.async_copy(q_hbm.at[dblk.at[jj]], qb[b], sem[b])
        pltpu.async_copy(k_hbm.at[sblk.at[jj]], kb[b], sem[b])
        pltpu.async_copy(v_hbm.at[sblk.at[jj]], vb[b], sem[b])

    def process(jj, b):
        pltpu.make_async_copy(q_hbm.at[pl.ds(0, C)], qb[b], sem[b]).wait()
        pltpu.make_async_copy(k_hbm.at[pl.ds(0, C)], kb[b], sem[b]).wait()
        pltpu.make_async_copy(v_hbm.at[pl.ds(0, C)], vb[b], sem[b]).wait()

        @plsc.parallel_loop(0, C // 2, 1, unroll=4)
        def pair(p2):
            e0 = 2 * p2
            e1 = e0 + 1
            o = (p2 & 7) * 256
            for h in range(H):
                prod[pl.ds(o + h * 16, 16)] = (
                    qb[b][e0, pl.ds(h * 16, 16)] * kb[b][e0, pl.ds(h * 16, 16)])
                prod[pl.ds(o + 128 + h * 16, 16)] = (
                    qb[b][e1, pl.ds(h * 16, 16)] * kb[b][e1, pl.ds(h * 16, 16)])
            idx0 = iota * 16 + o
            gs = [plsc.load_gather(prod, [idx0 + d]) for d in range(16)]
            while len(gs) > 1:
                gs = [gs[i] + gs[i + 1] for i in range(0, len(gs), 2)]
            alpha = jnp.exp(gs[0])
            for h in range(H):
                a0 = _take(alpha, jnp.full((16,), h, jnp.int32))
                msg[e0, pl.ds(h * 16, 16)] = vb[b][e0, pl.ds(h * 16, 16)] * a0
                a1 = _take(alpha, jnp.full((16,), 8 + h, jnp.int32))
                msg[e1, pl.ds(h * 16, 16)] = vb[b][e1, pl.ds(h * 16, 16)] * a1
            msg[e0, pl.ds(128, 16)] = alpha
            msg[e1, pl.ds(128, 16)] = _take(alpha, perm8)

        pltpu.sync_copy(msg, acc.at[dblk.at[jj]], add=True)

    def block(bi, carry):
        row0 = chunk_base + bi * IB
        pltpu.sync_copy(src_hbm.at[pl.ds(row0, IB)], sblk)
        pltpu.sync_copy(dst_hbm.at[pl.ds(row0, IB)], dblk)
        issue(0, 0)

        def chunk2(i, inner):
            j0 = 2 * i
            issue(j0 + 1, 1)
            process(j0, 0)
            issue(j0 + 2, 0)
            process(j0 + 1, 1)
            return inner

        lax.fori_loop(0, (IB - 1) // 2, chunk2, None)
        process(IB - 1, 0)
        return carry

    lax.fori_loop(0, NBLK, block, None)
    plsc.subcore_barrier()

    pltpu.sync_copy(acc.at[pl.ds(s * ROWS_PER_SUB, ROWS_PER_SUB)],
                    out_hbm.at[c, pl.ds(s * ROWS_PER_SUB, ROWS_PER_SUB)])


def _edge(q_t, k_t, v_t, src, dst):
    fn = functools.partial(
        pl.kernel,
        out_type=jax.ShapeDtypeStruct((NC, N_PAD, ACC_W), jnp.float32),
        mesh=plsc.VectorSubcoreMesh(core_axis_name="c", subcore_axis_name="s"),
        compiler_params=pltpu.CompilerParams(
            use_tc_tiling_on_sc=False, needs_layout_passes=False),
        scratch_types=(
            [pltpu.VMEM((IB, C), jnp.int32),
             pltpu.VMEM((IB, C), jnp.int32)]
            + [pltpu.VMEM((C, F), jnp.float32)] * 6
            + [
                pltpu.VMEM((C, ACC_W), jnp.float32),
                pltpu.VMEM((8 * 256,), jnp.float32),
                pltpu.VMEM_SHARED((N_PAD, ACC_W), jnp.float32),
                pltpu.SemaphoreType.DMA,
                pltpu.SemaphoreType.DMA,
            ]),
    )(_edge_body)
    return fn(q_t, k_t, v_t, src, dst)


def _post_body(relu, part_ref, x_ref, rp_ref, wa_ref, ba_ref, skip_ref, o_ref):
    a = part_ref[0] + part_ref[1]
    den = jnp.dot(a, rp_ref[...], preferred_element_type=jnp.float32) + 1e-16
    att = a[:, :F] / den
    g = jax.nn.gelu(att)
    o = jnp.dot(g, wa_ref[...], preferred_element_type=jnp.float32) + ba_ref[...]
    beta = jax.nn.sigmoid(skip_ref[0, 0])
    o = beta * o + (1.0 - beta) * x_ref[...]
    if relu:
        o = jnp.maximum(o, 0.0)
    o_ref[...] = o


_QPERM = np.zeros(F, np.int32)
for _h2 in range(4):
    for _w in range(32):
        _QPERM[32 * _h2 + _w] = 32 * _h2 + (
            _w // 2 if _w % 2 == 0 else 16 + (_w - 1) // 2)


def _post(partials, xin, wa, ba, skip, relu, blk=1000):
    rp = np.zeros((ACC_W, F), np.float32)
    for h in range(H):
        rp[F + h, h * D:(h + 1) * D] = 1.0
    return pl.pallas_call(
        functools.partial(_post_body, relu),
        grid=(N // blk,),
        in_specs=[
            pl.BlockSpec((NC, blk, ACC_W), lambda i: (0, i, 0)),
            pl.BlockSpec((blk, F), lambda i: (i, 0)),
            pl.BlockSpec((ACC_W, F), lambda i: (0, 0)),
            pl.BlockSpec((F, F), lambda i: (0, 0)),
            pl.BlockSpec((1, F), lambda i: (0, 0)),
            pl.BlockSpec((1, 1), lambda i: (0, 0)),
        ],
        out_specs=pl.BlockSpec((blk, F), lambda i: (i, 0)),
        out_shape=jax.ShapeDtypeStruct((N, F), jnp.float32),
    )(partials, xin, jnp.asarray(rp), wa, ba.reshape(1, F),
      skip.reshape(1, 1))


def _fold(p):
    scale = p["p_rel"] / np.sqrt(float(D))
    wq = p["Wq"].reshape(F, H, D) * scale[None, :, None]
    bq = p["bq"].reshape(H, D) * scale[:, None]
    wk = jnp.einsum("fhd,hde->fhe", p["Wk"].reshape(F, H, D), p["a_rel"])
    bk = jnp.einsum("hd,hde->he", p["bk"].reshape(H, D), p["a_rel"])
    wv = jnp.einsum("fhd,hde->fhe", p["Wv"].reshape(F, H, D), p["m_rel"])
    bv = jnp.einsum("hd,hde->he", p["bv"].reshape(H, D), p["m_rel"])
    w_all = jnp.concatenate(
        [wq.reshape(F, F), wk.reshape(F, F), wv.reshape(F, F)], axis=1)
    b_all = jnp.concatenate(
        [bq.reshape(F), bk.reshape(F), bv.reshape(F)]).reshape(1, 3 * F)
    return w_all, b_all


def kernel(x, edge_index, params):
    src = edge_index[0].astype(jnp.int32).reshape(E // C, C)
    dst = edge_index[1].astype(jnp.int32).reshape(E // C, C)
    h = x
    for li, name in enumerate(("l1", "l2")):
        p = params[name]
        w_all, b_all = _fold(p)
        q_t, k_t, v_t = _project(h, w_all, b_all)
        partials = _edge(q_t, k_t, v_t, src, dst)
        h = _post(partials, h, p["Wa"], p["ba"], p["skip"], relu=(li == 0))
    return h


# speedup vs baseline: 1.9539x; 1.3878x over previous
import functools

import jax
import jax.numpy as jnp
import numpy as np
from jax import lax
from jax.experimental import pallas as pl
from jax.experimental.pallas import tpu as pltpu
from jax.experimental.pallas import tpu_sc as plsc

N = 10000
E = 320000
F = 128
H = 8
D = 16
ACC_W = 144

NC = 2
NS = 16
NW = NC * NS
EW = E // NW
C = 40
NCHUNK = EW // C
IB = 25
NBLK = NCHUNK // IB
N_PAD = 10000
ROWS_PER_SUB = N_PAD // NS


def _proj_body(x_ref, w_ref, b_ref, q_ref, k_ref, v_ref):
    y = jnp.dot(x_ref[...], w_ref[...], preferred_element_type=jnp.float32)
    y = y + b_ref[...]
    q_ref[...] = y[:, :F]
    k_ref[...] = y[:, F:2 * F]
    v_ref[...] = y[:, 2 * F:]


def _project(xin, w_all, b_all, blk=1000):
    return pl.pallas_call(
        _proj_body,
        grid=(N // blk,),
        in_specs=[
            pl.BlockSpec((blk, F), lambda i: (i, 0)),
            pl.BlockSpec((F, 3 * F), lambda i: (0, 0)),
            pl.BlockSpec((1, 3 * F), lambda i: (0, 0)),
        ],
        out_specs=[pl.BlockSpec((blk, F), lambda i: (i, 0))] * 3,
        out_shape=[jax.ShapeDtypeStruct((N, F), jnp.float32)] * 3,
    )(xin, w_all, b_all)


_BITREV = [int(f"{i:04b}"[::-1], 2) for i in range(16)]

_DNUMS = lax.GatherDimensionNumbers(
    offset_dims=(), collapsed_slice_dims=(0,), start_index_map=(0,))


def _take(x, idx):
    return lax.gather(x, idx[:, None], _DNUMS, slice_sizes=(1,),
                      mode=lax.GatherScatterMode.PROMISE_IN_BOUNDS)


def _edge_body(q_hbm, k_hbm, v_hbm, src_hbm, dst_hbm, out_hbm,
               sblk, dblk, qb0, kb0, vb0, qb1, kb1, vb1,
               msg, prod, acc, sem0, sem1):
    c = lax.axis_index("c")
    s = lax.axis_index("s")
    wid = s * NC + c
    qb = (qb0, qb1)
    kb = (kb0, kb1)
    vb = (vb0, vb1)
    sem = (sem0, sem1)
    zvec = jnp.zeros((16,), jnp.float32)
    iota = lax.iota(jnp.int32, 16)
    perm8 = iota ^ 8
    base_l = (iota >> 3) * 128 + ((iota & 7) >> 1) * 32 + (iota & 1) * 8

    def zrow(r, carry):
        for c9 in range(ACC_W // 16):
            msg[r, pl.ds(c9 * 16, 16)] = zvec
        return carry
    lax.fori_loop(0, C, zrow, None)
    for j in range(ROWS_PER_SUB // C):
        pltpu.sync_copy(msg, acc.at[pl.ds(s * ROWS_PER_SUB + j * C, C)])
    pltpu.sync_copy(msg.at[pl.ds(0, ROWS_PER_SUB % C)],
                    acc.at[pl.ds(s * ROWS_PER_SUB + (ROWS_PER_SUB // C) * C,
                                 ROWS_PER_SUB % C)])
    plsc.subcore_barrier()

    chunk_base = wid * NCHUNK

    def issue(jj, b):
        pltpu.async_copy(q_hbm.at[dblk.at[jj]], qb[b], sem[b])
        pltpu.async_copy(k_hbm.at[sblk.at[jj]], kb[b], sem[b])
        pltpu.async_copy(v_hbm.at[sblk.at[jj]], vb[b], sem[b])

    def process(jj, b):
        pltpu.make_async_copy(q_hbm.at[pl.ds(0, C)], qb[b], sem[b]).wait()
        pltpu.make_async_copy(k_hbm.at[pl.ds(0, C)], kb[b], sem[b]).wait()
        pltpu.make_async_copy(v_hbm.at[pl.ds(0, C)], vb[b], sem[b]).wait()

        @plsc.parallel_loop(0, C // 2, 1, unroll=4)
        def pair(p2):
            e0 = 2 * p2
            e1 = e0 + 1
            pv = []
            for e in (e0, e1):
                for h in range(H):
                    pv.append(qb[b][e, pl.ds(h * 16, 16)]
                              * kb[b][e, pl.ds(h * 16, 16)])
            ps = [pv[_BITREV[i]] for i in range(16)]
            for kk in (8, 4, 2, 1):
                nxt = []
                for g in range(len(ps) // 2):
                    x, y = ps[2 * g], ps[2 * g + 1]
                    xf = x + _take(x, iota ^ kk)
                    yf = y + _take(y, iota ^ kk)
                    nxt.append(jnp.where((iota & kk) != 0, yf, xf))
                ps = nxt
            alpha = jnp.exp(ps[0])
            for h in range(H):
                a0 = _take(alpha, jnp.full((16,), h, jnp.int32))
                msg[e0, pl.ds(h * 16, 16)] = vb[b][e0, pl.ds(h * 16, 16)] * a0
                a1 = _take(alpha, jnp.full((16,), 8 + h, jnp.int32))
                msg[e1, pl.ds(h * 16, 16)] = vb[b][e1, pl.ds(h * 16, 16)] * a1
            msg[e0, pl.ds(128, 16)] = alpha
            msg[e1, pl.ds(128, 16)] = _take(alpha, perm8)

        pltpu.sync_copy(msg, acc.at[dblk.at[jj]], add=True)

    def block(bi, carry):
        row0 = chunk_base + bi * IB
        pltpu.sync_copy(src_hbm.at[pl.ds(row0, IB)], sblk)
        pltpu.sync_copy(dst_hbm.at[pl.ds(row0, IB)], dblk)
        issue(0, 0)

        def chunk2(i, inner):
            j0 = 2 * i
            issue(j0 + 1, 1)
            process(j0, 0)
            issue(j0 + 2, 0)
            process(j0 + 1, 1)
            return inner

        lax.fori_loop(0, (IB - 1) // 2, chunk2, None)
        process(IB - 1, 0)
        return carry

    lax.fori_loop(0, NBLK, block, None)
    plsc.subcore_barrier()

    pltpu.sync_copy(acc.at[pl.ds(s * ROWS_PER_SUB, ROWS_PER_SUB)],
                    out_hbm.at[c, pl.ds(s * ROWS_PER_SUB, ROWS_PER_SUB)])


def _edge(q_t, k_t, v_t, src, dst):
    fn = functools.partial(
        pl.kernel,
        out_type=jax.ShapeDtypeStruct((NC, N_PAD, ACC_W), jnp.float32),
        mesh=plsc.VectorSubcoreMesh(core_axis_name="c", subcore_axis_name="s"),
        compiler_params=pltpu.CompilerParams(
            use_tc_tiling_on_sc=False, needs_layout_passes=False),
        scratch_types=(
            [pltpu.VMEM((IB, C), jnp.int32),
             pltpu.VMEM((IB, C), jnp.int32)]
            + [pltpu.VMEM((C, F), jnp.float32)] * 6
            + [
                pltpu.VMEM((C, ACC_W), jnp.float32),
                pltpu.VMEM((8 * 256,), jnp.float32),
                pltpu.VMEM_SHARED((N_PAD, ACC_W), jnp.float32),
                pltpu.SemaphoreType.DMA,
                pltpu.SemaphoreType.DMA,
            ]),
    )(_edge_body)
    return fn(q_t, k_t, v_t, src, dst)


def _post_body(relu, part_ref, x_ref, rp_ref, wa_ref, ba_ref, skip_ref, o_ref):
    a = part_ref[0] + part_ref[1]
    den = jnp.dot(a, rp_ref[...], preferred_element_type=jnp.float32) + 1e-16
    att = a[:, :F] / den
    g = jax.nn.gelu(att)
    o = jnp.dot(g, wa_ref[...], preferred_element_type=jnp.float32) + ba_ref[...]
    beta = jax.nn.sigmoid(skip_ref[0, 0])
    o = beta * o + (1.0 - beta) * x_ref[...]
    if relu:
        o = jnp.maximum(o, 0.0)
    o_ref[...] = o


_QPERM = np.zeros(F, np.int32)
for _h2 in range(4):
    for _w in range(32):
        _QPERM[32 * _h2 + _w] = 32 * _h2 + (
            _w // 2 if _w % 2 == 0 else 16 + (_w - 1) // 2)


def _post(partials, xin, wa, ba, skip, relu, blk=1000):
    rp = np.zeros((ACC_W, F), np.float32)
    for h in range(H):
        rp[F + h, h * D:(h + 1) * D] = 1.0
    return pl.pallas_call(
        functools.partial(_post_body, relu),
        grid=(N // blk,),
        in_specs=[
            pl.BlockSpec((NC, blk, ACC_W), lambda i: (0, i, 0)),
            pl.BlockSpec((blk, F), lambda i: (i, 0)),
            pl.BlockSpec((ACC_W, F), lambda i: (0, 0)),
            pl.BlockSpec((F, F), lambda i: (0, 0)),
            pl.BlockSpec((1, F), lambda i: (0, 0)),
            pl.BlockSpec((1, 1), lambda i: (0, 0)),
        ],
        out_specs=pl.BlockSpec((blk, F), lambda i: (i, 0)),
        out_shape=jax.ShapeDtypeStruct((N, F), jnp.float32),
    )(partials, xin, jnp.asarray(rp), wa, ba.reshape(1, F),
      skip.reshape(1, 1))


def _fold(p):
    scale = p["p_rel"] / np.sqrt(float(D))
    wq = p["Wq"].reshape(F, H, D) * scale[None, :, None]
    bq = p["bq"].reshape(H, D) * scale[:, None]
    wk = jnp.einsum("fhd,hde->fhe", p["Wk"].reshape(F, H, D), p["a_rel"])
    bk = jnp.einsum("hd,hde->he", p["bk"].reshape(H, D), p["a_rel"])
    wv = jnp.einsum("fhd,hde->fhe", p["Wv"].reshape(F, H, D), p["m_rel"])
    bv = jnp.einsum("hd,hde->he", p["bv"].reshape(H, D), p["m_rel"])
    w_all = jnp.concatenate(
        [wq.reshape(F, F), wk.reshape(F, F), wv.reshape(F, F)], axis=1)
    b_all = jnp.concatenate(
        [bq.reshape(F), bk.reshape(F), bv.reshape(F)]).reshape(1, 3 * F)
    return w_all, b_all


def kernel(x, edge_index, params):
    src = edge_index[0].astype(jnp.int32).reshape(E // C, C)
    dst = edge_index[1].astype(jnp.int32).reshape(E // C, C)
    h = x
    for li, name in enumerate(("l1", "l2")):
        p = params[name]
        w_all, b_all = _fold(p)
        q_t, k_t, v_t = _project(h, w_all, b_all)
        partials = _edge(q_t, k_t, v_t, src, dst)
        h = _post(partials, h, p["Wa"], p["ba"], p["skip"], relu=(li == 0))
    return h
